# Initial kernel scaffold; baseline (speedup 1.0000x reference)
import jax
import jax.numpy as jnp
from jax.experimental import pallas as pl


def kernel(x, edge_index, edge_attr, We1, be1, We2, be2, xz_W0, xz_W1, xz_b, hz_W0, hz_W1, hz_b, xr_W0, xr_W1, xr_b, hr_W0, hr_W1, hr_b, xh_W0, xh_W1, xh_b, hh_W0, hh_W1, hh_b):
    raise NotImplementedError("write your pallas kernel here")


# speedup vs baseline: 11.1261x; 11.1261x over previous
import functools

import jax
import jax.numpy as jnp
from jax import lax
from jax.experimental import pallas as pl
from jax.experimental.pallas import tpu as pltpu
from jax.experimental.pallas import tpu_sc as plsc

N = 10000
E = 320000
D = 128
HID = 32

NC, NS = 2, 16
NW = NC * NS
CHUNK = 128
E_PAD = 327680
EROWS = E_PAD // CHUNK
ROWS_PER_W = EROWS // NW
GROUP = 8
NGROUP = ROWS_PER_W // GROUP
N_PAD = 10240
RPT = N_PAD // NS
DEGW = 16


def _deg_body(src_hbm, w_hbm, out_hbm, sidx, wv, upd, zbuf, acc):
    cid = lax.axis_index("c")
    sid = lax.axis_index("s")
    wid = cid * NS + sid
    row0 = wid * ROWS_PER_W
    zero16 = jnp.zeros((DEGW,), jnp.float32)

    @pl.loop(0, 128)
    def _zero(i):
        zbuf[i, :] = zero16

    for k in range(5):
        pltpu.sync_copy(zbuf, acc.at[pl.ds(sid * RPT + k * 128, 128)])
    plsc.subcore_barrier()

    @pl.loop(0, NGROUP)
    def _grp(g):
        r0 = row0 + g * GROUP
        pltpu.sync_copy(src_hbm.at[pl.ds(r0, GROUP)], sidx)
        pltpu.sync_copy(w_hbm.at[pl.ds(r0, GROUP)], wv)

        @pl.loop(0, GROUP)
        def _chunk(j):
            @pl.loop(0, CHUNK // 16)
            def _lanes(t):
                w16 = wv[j, pl.ds(t * 16, 16)]
                for l in range(16):
                    upd[t * 16 + l, :] = jnp.broadcast_to(w16[l], (DEGW,))

            pltpu.sync_copy(upd, acc.at[sidx.at[j]], add=True)

    plsc.subcore_barrier()
    pltpu.sync_copy(acc.at[pl.ds(sid * RPT, RPT)],
                    out_hbm.at[cid, pl.ds(sid * RPT, RPT)])


_deg_call = pl.kernel(
    _deg_body,
    out_type=jax.ShapeDtypeStruct((NC, N_PAD, DEGW), jnp.float32),
    mesh=plsc.VectorSubcoreMesh(core_axis_name="c", subcore_axis_name="s"),
    compiler_params=pltpu.CompilerParams(use_tc_tiling_on_sc=False),
    scratch_types=[
        pltpu.VMEM((GROUP, CHUNK), jnp.int32),
        pltpu.VMEM((GROUP, CHUNK), jnp.float32),
        pltpu.VMEM((CHUNK, DEGW), jnp.float32),
        pltpu.VMEM((128, DEGW), jnp.float32),
        pltpu.VMEM_SHARED((N_PAD, DEGW), jnp.float32),
    ],
)


def _acc_body(y2_hbm, src_hbm, dst_hbm, w_hbm, out_hbm,
              sidx, didx, wv, rows, zbuf, sem, acc):
    cid = lax.axis_index("c")
    sid = lax.axis_index("s")
    wid = cid * NS + sid
    row0 = wid * ROWS_PER_W
    zero16 = jnp.zeros((16,), jnp.float32)

    @pl.loop(0, 128)
    def _zero(i):
        for jj in range(4):
            zbuf[i, pl.ds(jj * 16, 16)] = zero16

    for k in range(5):
        pltpu.sync_copy(zbuf, acc.at[pl.ds(sid * RPT + k * 128, 128)])
    plsc.subcore_barrier()

    @pl.loop(0, NGROUP)
    def _grp(g):
        r0 = row0 + g * GROUP
        pltpu.sync_copy(src_hbm.at[pl.ds(r0, GROUP)], sidx)
        pltpu.sync_copy(dst_hbm.at[pl.ds(r0, GROUP)], didx)
        pltpu.sync_copy(w_hbm.at[pl.ds(r0, GROUP)], wv)

        @pl.loop(0, GROUP)
        def _chunk(j):
            pltpu.async_copy(y2_hbm.at[sidx.at[j]], rows, sem).wait()

            @pl.loop(0, CHUNK // 16)
            def _lanes(t):
                w16 = wv[j, pl.ds(t * 16, 16)]
                for l in range(16):
                    wi = w16[l]
                    i = t * 16 + l
                    for jj in range(4):
                        rows[i, pl.ds(jj * 16, 16)] = rows[i, pl.ds(jj * 16, 16)] * wi

            pltpu.sync_copy(rows, acc.at[didx.at[j]], add=True)

    plsc.subcore_barrier()
    pltpu.sync_copy(acc.at[pl.ds(sid * RPT, RPT)],
                    out_hbm.at[cid, pl.ds(sid * RPT, RPT)])


_acc_call = pl.kernel(
    _acc_body,
    out_type=jax.ShapeDtypeStruct((NC, N_PAD, 64), jnp.float32),
    mesh=plsc.VectorSubcoreMesh(core_axis_name="c", subcore_axis_name="s"),
    compiler_params=pltpu.CompilerParams(use_tc_tiling_on_sc=False),
    scratch_types=[
        pltpu.VMEM((GROUP, CHUNK), jnp.int32),
        pltpu.VMEM((GROUP, CHUNK), jnp.int32),
        pltpu.VMEM((GROUP, CHUNK), jnp.float32),
        pltpu.VMEM((CHUNK, 64), jnp.float32),
        pltpu.VMEM((128, 64), jnp.float32),
        pltpu.SemaphoreType.DMA,
        pltpu.VMEM_SHARED((N_PAD, 64), jnp.float32),
    ],
)


def _mlp_body(ea_ref, w1_ref, b1_ref, w2_ref, b2_ref, o_ref):
    h = jnp.dot(ea_ref[...], w1_ref[...], preferred_element_type=jnp.float32)
    h = jnp.maximum(h + b1_ref[...], 0.0)
    o = jnp.dot(h, w2_ref[...], preferred_element_type=jnp.float32) + b2_ref[...]
    o_ref[...] = jax.nn.sigmoid(o)


def _edge_mlp(edge_attr, We1, be1, We2, be2):
    be = 4000
    grid = E // be
    return pl.pallas_call(
        _mlp_body,
        grid=(grid,),
        in_specs=[
            pl.BlockSpec((be, 16), lambda i: (i, 0)),
            pl.BlockSpec((16, 32), lambda i: (0, 0)),
            pl.BlockSpec((1, 32), lambda i: (0, 0)),
            pl.BlockSpec((32, 1), lambda i: (0, 0)),
            pl.BlockSpec((1, 1), lambda i: (0, 0)),
        ],
        out_specs=pl.BlockSpec((be, 1), lambda i: (i, 0)),
        out_shape=jax.ShapeDtypeStruct((E, 1), jnp.float32),
    )(edge_attr, We1, be1, We2, be2)


def _proj_body(x_ref, w_ref, o_ref):
    o_ref[...] = jnp.dot(x_ref[...], w_ref[...], preferred_element_type=jnp.float32)


def _proj(x, wcat):
    bn = 2000
    return pl.pallas_call(
        _proj_body,
        grid=(N // bn,),
        in_specs=[
            pl.BlockSpec((bn, D), lambda i: (i, 0)),
            pl.BlockSpec((D, D), lambda i: (0, 0)),
        ],
        out_specs=pl.BlockSpec((bn, D), lambda i: (i, 0)),
        out_shape=jax.ShapeDtypeStruct((N, D), jnp.float32),
    )(x, wcat)


def _dis_body(degp_ref, y_ref, dis_ref, y2_ref):
    d = degp_ref[0, :, 0:1] + degp_ref[1, :, 0:1]
    dis = jnp.where(d > 0, lax.rsqrt(jnp.where(d > 0, d, 1.0)), 0.0)
    dis_ref[...] = dis
    y2_ref[...] = dis * y_ref[:, :64]


def _dis_y2(degp, xall):
    bn = 1024
    return pl.pallas_call(
        _dis_body,
        grid=(N_PAD // bn,),
        in_specs=[
            pl.BlockSpec((NC, bn, DEGW), lambda i: (0, i, 0)),
            pl.BlockSpec((bn, D), lambda i: (i, 0)),
        ],
        out_specs=[
            pl.BlockSpec((bn, 1), lambda i: (i, 0)),
            pl.BlockSpec((bn, 64), lambda i: (i, 0)),
        ],
        out_shape=[
            jax.ShapeDtypeStruct((N_PAD, 1), jnp.float32),
            jax.ShapeDtypeStruct((N_PAD, 64), jnp.float32),
        ],
    )(degp, xall)


def _fin_body(sacc_ref, dis_ref, xall_ref, bz_ref, bh_ref, o_ref):
    s = sacc_ref[0] + sacc_ref[1]
    t = -dis_ref[...] * s
    z = jax.nn.sigmoid(xall_ref[:, 64:96] + t[:, :32] + bz_ref[...])
    ht = jnp.tanh(xall_ref[:, 96:128] + t[:, 32:] + bh_ref[...])
    o_ref[...] = (1.0 - z) * ht


def _final(sacc, dis, xall, bz, bh):
    bn = 1024
    return pl.pallas_call(
        _fin_body,
        grid=(N_PAD // bn,),
        in_specs=[
            pl.BlockSpec((NC, bn, 64), lambda i: (0, i, 0)),
            pl.BlockSpec((bn, 1), lambda i: (i, 0)),
            pl.BlockSpec((bn, D), lambda i: (i, 0)),
            pl.BlockSpec((1, HID), lambda i: (0, 0)),
            pl.BlockSpec((1, HID), lambda i: (0, 0)),
        ],
        out_specs=pl.BlockSpec((bn, HID), lambda i: (i, 0)),
        out_shape=jax.ShapeDtypeStruct((N_PAD, HID), jnp.float32),
    )(sacc, dis, xall, bz, bh)


def kernel(x, edge_index, edge_attr, We1, be1, We2, be2,
           xz_W0, xz_W1, xz_b, hz_W0, hz_W1, hz_b,
           xr_W0, xr_W1, xr_b, hr_W0, hr_W1, hr_b,
           xh_W0, xh_W1, xh_b, hh_W0, hh_W1, hh_b):
    src = edge_index[0]
    dst = edge_index[1]

    ew = _edge_mlp(edge_attr, We1, be1.reshape(1, 32), We2, be2.reshape(1, 1))
    w = jnp.where(src == dst, 0.0, ew[:, 0])

    pad = E_PAD - E
    src2 = jnp.concatenate([src, jnp.zeros((pad,), jnp.int32)]).reshape(EROWS, CHUNK)
    dst2 = jnp.concatenate([dst, jnp.zeros((pad,), jnp.int32)]).reshape(EROWS, CHUNK)
    w2 = jnp.concatenate([w, jnp.zeros((pad,), jnp.float32)]).reshape(EROWS, CHUNK)

    wcat = jnp.concatenate([xz_W1, xh_W1, xz_W0, xh_W0], axis=1)
    xall = _proj(x, wcat)

    xall_pad = jnp.pad(xall, ((0, N_PAD - N), (0, 0)))

    degp = _deg_call(src2, w2)
    dis, y2 = _dis_y2(degp, xall_pad)
    sacc = _acc_call(y2, src2, dst2, w2)

    bz = (xz_b + hz_b).reshape(1, HID)
    bh = (xh_b + hh_b).reshape(1, HID)
    return _final(sacc, dis, xall_pad, bz, bh)[:N]


# speedup vs baseline: 17.2910x; 1.5541x over previous
import functools

import jax
import jax.numpy as jnp
from jax import lax
from jax.experimental import pallas as pl
from jax.experimental.pallas import tpu as pltpu
from jax.experimental.pallas import tpu_sc as plsc

N = 10000
E = 320000
D = 128
HID = 32

NC, NS = 2, 16
NW = NC * NS
CHUNK = 128
E_PAD = 327680
EROWS = E_PAD // CHUNK
ROWS_PER_W = EROWS // NW
GROUP = 8
NGROUP = ROWS_PER_W // GROUP
N_PAD = 10240
RPT = N_PAD // NS
DEGW = 16


def _deg_body(src_hbm, w_hbm, out_hbm, sidx, wv, upd, zbuf,
              s0, s1, s2, s3, acc):
    cid = lax.axis_index("c")
    sid = lax.axis_index("s")
    wid = cid * NS + sid
    row0 = wid * ROWS_PER_W
    ssem = [s0, s1, s2, s3]
    zero16 = jnp.zeros((DEGW,), jnp.float32)

    @pl.loop(0, 128)
    def _zero(i):
        zbuf[i, :] = zero16

    for k in range(5):
        pltpu.sync_copy(zbuf, acc.at[pl.ds(sid * RPT + k * 128, 128)])
    plsc.subcore_barrier()

    pltpu.sync_copy(src_hbm.at[pl.ds(row0, ROWS_PER_W)], sidx)
    pltpu.sync_copy(w_hbm.at[pl.ds(row0, ROWS_PER_W)], wv)

    def _build(j, b):
        @pl.loop(0, CHUNK // 16)
        def _lanes(t):
            w16 = wv[j, pl.ds(t * 16, 16)]
            for l in range(16):
                upd[b, t * 16 + l, :] = jnp.broadcast_to(w16[l], (DEGW,))

    def _scat(j, b):
        pltpu.make_async_copy(upd.at[b], acc.at[sidx.at[j]], ssem[b]).start(add=True)

    def _swait(j, b):
        pltpu.make_async_copy(upd.at[b], acc.at[sidx.at[j]], ssem[b]).wait()

    for b in range(4):
        _build(b, b)
        _scat(b, b)

    @pl.loop(0, (ROWS_PER_W - 4) // 4)
    def _grp(g):
        for b in range(4):
            j = 4 + g * 4 + b
            _swait(j, b)
            _build(j, b)
            _scat(j, b)

    for b in range(4):
        _swait(0, b)
    plsc.subcore_barrier()
    pltpu.sync_copy(acc.at[pl.ds(sid * RPT, RPT)],
                    out_hbm.at[cid, pl.ds(sid * RPT, RPT)])


_deg_call = pl.kernel(
    _deg_body,
    out_type=jax.ShapeDtypeStruct((NC, N_PAD, DEGW), jnp.float32),
    mesh=plsc.VectorSubcoreMesh(core_axis_name="c", subcore_axis_name="s"),
    compiler_params=pltpu.CompilerParams(use_tc_tiling_on_sc=False),
    scratch_types=[
        pltpu.VMEM((ROWS_PER_W, CHUNK), jnp.int32),
        pltpu.VMEM((ROWS_PER_W, CHUNK), jnp.float32),
        pltpu.VMEM((4, CHUNK, DEGW), jnp.float32),
        pltpu.VMEM((128, DEGW), jnp.float32),
        pltpu.SemaphoreType.DMA,
        pltpu.SemaphoreType.DMA,
        pltpu.SemaphoreType.DMA,
        pltpu.SemaphoreType.DMA,
        pltpu.VMEM_SHARED((N_PAD, DEGW), jnp.float32),
    ],
)


def _acc_body(y2_hbm, src_hbm, dst_hbm, w_hbm, out_hbm,
              sidx, didx, wv, rows, zbuf,
              g0, g1, g2, g3, s0, s1, s2, s3, acc):
    cid = lax.axis_index("c")
    sid = lax.axis_index("s")
    wid = cid * NS + sid
    row0 = wid * ROWS_PER_W
    gsem = [g0, g1, g2, g3]
    ssem = [s0, s1, s2, s3]
    zero16 = jnp.zeros((16,), jnp.float32)

    @pl.loop(0, 128)
    def _zero(i):
        for jj in range(4):
            zbuf[i, pl.ds(jj * 16, 16)] = zero16

    for k in range(5):
        pltpu.sync_copy(zbuf, acc.at[pl.ds(sid * RPT + k * 128, 128)])
    plsc.subcore_barrier()

    pltpu.sync_copy(src_hbm.at[pl.ds(row0, ROWS_PER_W)], sidx)
    pltpu.sync_copy(dst_hbm.at[pl.ds(row0, ROWS_PER_W)], didx)
    pltpu.sync_copy(w_hbm.at[pl.ds(row0, ROWS_PER_W)], wv)

    def _gstart(j, b):
        pltpu.make_async_copy(y2_hbm.at[sidx.at[j]], rows.at[b], gsem[b]).start()

    def _gwait(j, b):
        pltpu.make_async_copy(y2_hbm.at[sidx.at[j]], rows.at[b], gsem[b]).wait()

    def _scale(j, b):
        @pl.loop(0, CHUNK // 16)
        def _lanes(t):
            w16 = wv[j, pl.ds(t * 16, 16)]
            for l in range(16):
                wi = w16[l]
                i = t * 16 + l
                for jj in range(4):
                    rows[b, i, pl.ds(jj * 16, 16)] = rows[b, i, pl.ds(jj * 16, 16)] * wi

    def _sstart(j, b):
        pltpu.make_async_copy(rows.at[b], acc.at[didx.at[j]], ssem[b]).start(add=True)

    def _swait(j, b):
        pltpu.make_async_copy(rows.at[b], acc.at[didx.at[j]], ssem[b]).wait()

    _gstart(0, 0)
    _gstart(1, 1)
    for j in (0, 1):
        _gstart(j + 2, (j + 2) % 4)
        _gwait(j, j % 4)
        _scale(j, j % 4)
        _sstart(j, j % 4)

    @pl.loop(0, (ROWS_PER_W - 4) // 4)
    def _grp(g):
        for bp in range(4):
            j = 2 + g * 4 + bp
            b = (2 + bp) % 4
            b2 = bp
            _swait(j, b2)
            _gstart(j + 2, b2)
            _gwait(j, b)
            _scale(j, b)
            _sstart(j, b)

    for j in (ROWS_PER_W - 2, ROWS_PER_W - 1):
        b = j % 4
        _swait(j, (j + 2) % 4)
        _gwait(j, b)
        _scale(j, b)
        _sstart(j, b)
    for b in ((ROWS_PER_W - 2) % 4, (ROWS_PER_W - 1) % 4):
        _swait(0, b)
    plsc.subcore_barrier()
    pltpu.sync_copy(acc.at[pl.ds(sid * RPT, RPT)],
                    out_hbm.at[cid, pl.ds(sid * RPT, RPT)])


_acc_call = pl.kernel(
    _acc_body,
    out_type=jax.ShapeDtypeStruct((NC, N_PAD, 64), jnp.float32),
    mesh=plsc.VectorSubcoreMesh(core_axis_name="c", subcore_axis_name="s"),
    compiler_params=pltpu.CompilerParams(use_tc_tiling_on_sc=False),
    scratch_types=[
        pltpu.VMEM((ROWS_PER_W, CHUNK), jnp.int32),
        pltpu.VMEM((ROWS_PER_W, CHUNK), jnp.int32),
        pltpu.VMEM((ROWS_PER_W, CHUNK), jnp.float32),
        pltpu.VMEM((4, CHUNK, 64), jnp.float32),
        pltpu.VMEM((128, 64), jnp.float32),
        pltpu.SemaphoreType.DMA,
        pltpu.SemaphoreType.DMA,
        pltpu.SemaphoreType.DMA,
        pltpu.SemaphoreType.DMA,
        pltpu.SemaphoreType.DMA,
        pltpu.SemaphoreType.DMA,
        pltpu.SemaphoreType.DMA,
        pltpu.SemaphoreType.DMA,
        pltpu.VMEM_SHARED((N_PAD, 64), jnp.float32),
    ],
)


def _mlp_body(ea_ref, w1_ref, b1_ref, w2_ref, b2_ref, o_ref):
    h = jnp.dot(ea_ref[...], w1_ref[...], preferred_element_type=jnp.float32)
    h = jnp.maximum(h + b1_ref[...], 0.0)
    o = jnp.dot(h, w2_ref[...], preferred_element_type=jnp.float32) + b2_ref[...]
    o_ref[...] = jax.nn.sigmoid(o)


def _edge_mlp(ea8, w1b, b1b, w2b, b2b):
    br = 2000
    rows8 = E // 8
    return pl.pallas_call(
        _mlp_body,
        grid=(rows8 // br,),
        in_specs=[
            pl.BlockSpec((br, D), lambda i: (i, 0)),
            pl.BlockSpec((D, 256), lambda i: (0, 0)),
            pl.BlockSpec((1, 256), lambda i: (0, 0)),
            pl.BlockSpec((256, 8), lambda i: (0, 0)),
            pl.BlockSpec((1, 8), lambda i: (0, 0)),
        ],
        out_specs=pl.BlockSpec((br, 8), lambda i: (i, 0)),
        out_shape=jax.ShapeDtypeStruct((rows8, 8), jnp.float32),
    )(ea8, w1b, b1b, w2b, b2b)


def _proj_body(x_ref, w_ref, o_ref):
    o_ref[...] = jnp.dot(x_ref[...], w_ref[...], preferred_element_type=jnp.float32)


def _proj(x, wcat):
    bn = 2000
    return pl.pallas_call(
        _proj_body,
        grid=(N // bn,),
        in_specs=[
            pl.BlockSpec((bn, D), lambda i: (i, 0)),
            pl.BlockSpec((D, D), lambda i: (0, 0)),
        ],
        out_specs=pl.BlockSpec((bn, D), lambda i: (i, 0)),
        out_shape=jax.ShapeDtypeStruct((N, D), jnp.float32),
    )(x, wcat)


def _dis_body(degp_ref, y_ref, dis_ref, y2_ref):
    d = degp_ref[0, :, 0:1] + degp_ref[1, :, 0:1]
    dis = jnp.where(d > 0, lax.rsqrt(jnp.where(d > 0, d, 1.0)), 0.0)
    dis_ref[...] = dis
    y2_ref[...] = dis * y_ref[:, :64]


def _dis_y2(degp, xall):
    bn = 1024
    return pl.pallas_call(
        _dis_body,
        grid=(N_PAD // bn,),
        in_specs=[
            pl.BlockSpec((NC, bn, DEGW), lambda i: (0, i, 0)),
            pl.BlockSpec((bn, D), lambda i: (i, 0)),
        ],
        out_specs=[
            pl.BlockSpec((bn, 1), lambda i: (i, 0)),
            pl.BlockSpec((bn, 64), lambda i: (i, 0)),
        ],
        out_shape=[
            jax.ShapeDtypeStruct((N_PAD, 1), jnp.float32),
            jax.ShapeDtypeStruct((N_PAD, 64), jnp.float32),
        ],
    )(degp, xall)


def _fin_body(sacc_ref, dis_ref, xall_ref, bz_ref, bh_ref, o_ref):
    s = sacc_ref[0] + sacc_ref[1]
    t = -dis_ref[...] * s
    z = jax.nn.sigmoid(xall_ref[:, 64:96] + t[:, :32] + bz_ref[...])
    ht = jnp.tanh(xall_ref[:, 96:128] + t[:, 32:] + bh_ref[...])
    o_ref[...] = (1.0 - z) * ht


def _final(sacc, dis, xall, bz, bh):
    bn = 1024
    return pl.pallas_call(
        _fin_body,
        grid=(N_PAD // bn,),
        in_specs=[
            pl.BlockSpec((NC, bn, 64), lambda i: (0, i, 0)),
            pl.BlockSpec((bn, 1), lambda i: (i, 0)),
            pl.BlockSpec((bn, D), lambda i: (i, 0)),
            pl.BlockSpec((1, HID), lambda i: (0, 0)),
            pl.BlockSpec((1, HID), lambda i: (0, 0)),
        ],
        out_specs=pl.BlockSpec((bn, HID), lambda i: (i, 0)),
        out_shape=jax.ShapeDtypeStruct((N_PAD, HID), jnp.float32),
    )(sacc, dis, xall, bz, bh)


def kernel(x, edge_index, edge_attr, We1, be1, We2, be2,
           xz_W0, xz_W1, xz_b, hz_W0, hz_W1, hz_b,
           xr_W0, xr_W1, xr_b, hr_W0, hr_W1, hr_b,
           xh_W0, xh_W1, xh_b, hh_W0, hh_W1, hh_b):
    src = edge_index[0]
    dst = edge_index[1]

    ea8 = edge_attr.reshape(E // 8, 128)
    eye8 = jnp.eye(8, dtype=jnp.float32)
    w1b = jnp.kron(eye8, We1)
    w2b = jnp.kron(eye8, We2)
    b1b = jnp.tile(be1, 8).reshape(1, 256)
    b2b = jnp.tile(be2, 8).reshape(1, 8)
    ew8 = _edge_mlp(ea8, w1b, b1b, w2b, b2b)
    w8 = jnp.where(src.reshape(E // 8, 8) == dst.reshape(E // 8, 8), 0.0, ew8)

    padr = EROWS - E // CHUNK
    src2 = jnp.pad(src.reshape(E // CHUNK, CHUNK), ((0, padr), (0, 0)))
    dst2 = jnp.pad(dst.reshape(E // CHUNK, CHUNK), ((0, padr), (0, 0)))
    w2 = jnp.pad(w8.reshape(E // CHUNK, CHUNK), ((0, padr), (0, 0)))

    wcat = jnp.concatenate([xz_W1, xh_W1, xz_W0, xh_W0], axis=1)
    xall = _proj(x, wcat)

    xall_pad = jnp.pad(xall, ((0, N_PAD - N), (0, 0)))

    degp = _deg_call(src2, w2)
    dis, y2 = _dis_y2(degp, xall_pad)
    sacc = _acc_call(y2, src2, dst2, w2)

    bz = (xz_b + hz_b).reshape(1, HID)
    bh = (xh_b + hh_b).reshape(1, HID)
    return _final(sacc, dis, xall_pad, bz, bh)[:N]
